# baseline (speedup 1.0000x reference)
import jax
import jax.numpy as jnp
from jax import lax
from jax.experimental import pallas as pl
from jax.experimental.pallas import tpu as pltpu
from jax.experimental.pallas import tpu_sc as plsc

F32 = jnp.float32
I32 = jnp.int32
NC, NS, LANES = 2, 16, 16
NW = NC * NS
H = 64
CH = 80


_SC_PARAMS = pltpu.CompilerParams(needs_layout_passes=False,
                                  use_tc_tiling_on_sc=False)


def _mesh():
    return plsc.VectorSubcoreMesh(core_axis_name="c", subcore_axis_name="s",
                                  num_cores=NC, num_subcores=NS)


def _wid():
    return lax.axis_index("s") * NC + lax.axis_index("c")


def _geom_sc(E, N):
    EPT = E // NW
    NCHUNK = EPT // CH
    GROUPS = CH // LANES

    def body(x16, row2, col2, rp3, ri_v, ci_v, xr_v, xc_v, rp_v, sems):
        wid = _wid()
        pltpu.sync_copy(row2.at[pl.ds(wid * NCHUNK, NCHUNK)], ri_v)
        pltpu.sync_copy(col2.at[pl.ds(wid * NCHUNK, NCHUNK)], ci_v)

        def issue(i, b):
            cr = pltpu.async_copy(x16.at[ri_v.at[i]], xr_v.at[b], sems.at[b])
            cc = pltpu.async_copy(x16.at[ci_v.at[i]], xc_v.at[b],
                                  sems.at[2 + b])
            return cr, cc

        def wait(i, b):
            pltpu.make_async_copy(x16.at[ri_v.at[i]], xr_v.at[b],
                                  sems.at[b]).wait()
            pltpu.make_async_copy(x16.at[ci_v.at[i]], xc_v.at[b],
                                  sems.at[2 + b]).wait()

        def work(i, b):
            for g in range(GROUPS):
                eids = lax.iota(I32, LANES) + g * LANES
                for j in range(3):
                    jv = jnp.full((LANES,), j, I32)
                    d = (plsc.load_gather(xr_v.at[b], [eids, jv])
                         - plsc.load_gather(xc_v.at[b], [eids, jv]))
                    rp_v[j, pl.ds(g * LANES, LANES)] = d
            base = wid * EPT + i * CH
            pltpu.sync_copy(rp_v.at[0], rp3.at[0, pl.ds(base, CH)])
            pltpu.sync_copy(rp_v.at[1], rp3.at[1, pl.ds(base, CH)])
            pltpu.sync_copy(rp_v.at[2], rp3.at[2, pl.ds(base, CH)])

        issue(0, 0)

        def pair(p, _):
            i0 = 2 * p
            issue(i0 + 1, 1)
            wait(i0, 0)
            work(i0, 0)

            @pl.when(i0 + 2 < NCHUNK)
            def _():
                issue(i0 + 2, 0)
            wait(i0 + 1, 1)
            work(i0 + 1, 1)
            return 0

        lax.fori_loop(0, NCHUNK // 2, pair, 0)
        if NCHUNK % 2:
            wait(NCHUNK - 1, 0)
            work(NCHUNK - 1, 0)

    out = jax.ShapeDtypeStruct((3, E), F32)
    scratch = [
        pltpu.VMEM((NCHUNK, CH), I32), pltpu.VMEM((NCHUNK, CH), I32),
        pltpu.VMEM((2, CH, 16), F32), pltpu.VMEM((2, CH, 16), F32),
        pltpu.VMEM((3, CH), F32),
        pltpu.SemaphoreType.DMA((4,)),
    ]
    return pl.kernel(body, out_type=out, mesh=_mesh(), scratch_types=scratch,
                     compiler_params=_SC_PARAMS)


def _layer_sc(E, N, extras, write_t3):
    EPT = E // NW
    NCHUNK = EPT // CH
    SB = 25
    DEPTH = 2
    W = 64 + (16 if extras else 0)
    ROWS_PT = N // NS
    ZR = 125

    def body(*refs):
        it = iter(refs)
        A = next(it); Bt = next(it); row2 = next(it); col2 = next(it)
        rd2 = next(it); w1c = next(it)
        out = next(it)
        ri_v = next(it); ci_v = next(it); rd_v = next(it); w1c_v = next(it)
        a_v = next(it); b_v = next(it); pay_v = next(it)
        sems = next(it)
        if not write_t3:
            S_sh = next(it); zbuf = next(it)

        cid = lax.axis_index("c")
        sid = lax.axis_index("s")
        wid = _wid()
        pltpu.sync_copy(w1c, w1c_v)

        if not write_t3:
            def zrow(i, _):
                for r in range(W // 16):
                    zbuf[i, pl.ds(r * 16, 16)] = jnp.zeros((16,), F32)
                return 0
            lax.fori_loop(0, ZR, zrow, 0)
            for q in range(ROWS_PT // ZR):
                pltpu.sync_copy(
                    zbuf, S_sh.at[pl.ds(sid * ROWS_PT + q * ZR, ZR)])
            plsc.subcore_barrier()

        def issue(i, b):
            pltpu.async_copy(A.at[ri_v.at[i]], a_v.at[b], sems.at[b])
            pltpu.async_copy(Bt.at[ci_v.at[i]], b_v.at[b], sems.at[4 + b])

        def wait(i, b):
            pltpu.make_async_copy(A.at[ri_v.at[i]], a_v.at[b],
                                  sems.at[b]).wait()
            pltpu.make_async_copy(Bt.at[ci_v.at[i]], b_v.at[b],
                                  sems.at[4 + b]).wait()

        def out_dst(sb, i, pb):
            if write_t3:
                base = wid * EPT + (sb * SB + i) * CH
                return out.at[pl.ds(base, CH)]
            return S_sh.at[ri_v.at[i]]

        def work(sb, i, b, pb):
            @pl.when(i >= 2)
            def _():
                pltpu.make_async_copy(pay_v.at[pb], out_dst(sb, i, pb),
                                      sems.at[8 + pb]).wait()

            if extras:
                one0 = jnp.where(lax.iota(I32, 16) == 0, 1.0, 0.0)

            def group(g, _):
                rd16 = rd_v[i, pl.ds(g * 16, 16)]
                for j in range(16):
                    rdv = rd16[j]
                    for r in range(4):
                        sl = pl.ds(r * 16, 16)
                        pay_v[pb, g * 16 + j, sl] = jnp.maximum(
                            a_v[b, g * 16 + j, sl] + b_v[b, g * 16 + j, sl]
                            + rdv * w1c_v[sl], 0.0)
                    if extras:
                        pay_v[pb, g * 16 + j, pl.ds(64, 16)] = one0
                return 0

            lax.fori_loop(0, CH // 16, group, 0)
            pltpu.async_copy(pay_v.at[pb], out_dst(sb, i, pb),
                             sems.at[8 + pb], add=not write_t3)

        def sblock(sb, _):
            blk = pl.ds(wid * NCHUNK + sb * SB, SB)
            pltpu.sync_copy(row2.at[blk], ri_v)
            pltpu.sync_copy(col2.at[blk], ci_v)
            pltpu.sync_copy(rd2.at[blk], rd_v)
            for d in range(DEPTH - 1):
                issue(d, d)

            def turn(q, _):
                for l in range(DEPTH):
                    i = DEPTH * q + l

                    @pl.when(i + DEPTH - 1 < SB)
                    def _():
                        issue(i + DEPTH - 1, (l + DEPTH - 1) % DEPTH)
                    wait(i, l)
                    work(sb, i, l, l % 2)
                return 0

            lax.fori_loop(0, SB // DEPTH, turn, 0)
            for i in range(SB - SB % DEPTH, SB):
                wait(i, i % DEPTH)
                work(sb, i, i % DEPTH, i % 2)
            pltpu.make_async_copy(pay_v.at[(SB - 1) % 2],
                                  out_dst(sb, SB - 1, (SB - 1) % 2),
                                  sems.at[8 + (SB - 1) % 2]).wait()
            pltpu.make_async_copy(pay_v.at[(SB - 2) % 2],
                                  out_dst(sb, SB - 2, (SB - 2) % 2),
                                  sems.at[8 + (SB - 2) % 2]).wait()
            return 0

        lax.fori_loop(0, NCHUNK // SB, sblock, 0)

        if not write_t3:
            plsc.subcore_barrier()
            for q in range(ROWS_PT // ZR):
                rows = pl.ds(sid * ROWS_PT + q * ZR, ZR)
                pltpu.sync_copy(S_sh.at[rows], out.at[cid, rows])

    if write_t3:
        out = jax.ShapeDtypeStruct((E, 64), F32)
    else:
        out = jax.ShapeDtypeStruct((NC, N, W), F32)
    scratch = [
        pltpu.VMEM((SB, CH), I32), pltpu.VMEM((SB, CH), I32),
        pltpu.VMEM((SB, CH), F32), pltpu.VMEM((64,), F32),
    ]
    scratch += [
        pltpu.VMEM((DEPTH, CH, 64), F32), pltpu.VMEM((DEPTH, CH, 64), F32),
        pltpu.VMEM((2, CH, W), F32),
        pltpu.SemaphoreType.DMA((10,)),
    ]
    if not write_t3:
        scratch += [pltpu.VMEM_SHARED((N, W), F32), pltpu.VMEM((ZR, W), F32)]
    return pl.kernel(body, out_type=out, mesh=_mesh(), scratch_types=scratch,
                     compiler_params=_SC_PARAMS)


def _final_sc(E, N, NK):
    EPT = E // NW
    NCHUNK = EPT // CH
    SB = 25
    W = 64 * NK
    ROWS_PT = N // NS
    ZR = 25

    def body(*refs):
        it = iter(refs)
        u = next(it)
        row2 = next(it)
        svec = [next(it) for _ in range(NK)]
        out = next(it)
        ri_v = next(it)
        sv_v = [next(it) for _ in range(NK)]
        u_v = next(it)
        pay_v = next(it)
        sems = next(it)
        S_sh = next(it)
        zbuf = next(it)
        cid = lax.axis_index("c")
        sid = lax.axis_index("s")
        wid = _wid()

        def zrow(i, _):
            for r in range(W // 16):
                zbuf[i, pl.ds(r * 16, 16)] = jnp.zeros((16,), F32)
            return 0
        lax.fori_loop(0, ZR, zrow, 0)
        for q in range(ROWS_PT // ZR):
            pltpu.sync_copy(zbuf, S_sh.at[pl.ds(sid * ROWS_PT + q * ZR, ZR)])
        plsc.subcore_barrier()

        def issue(sb, i, b):
            base = wid * EPT + (sb * SB + i) * CH
            pltpu.async_copy(u.at[pl.ds(base, CH)], u_v.at[b], sems.at[b])

        def wait(sb, i, b):
            base = wid * EPT + (sb * SB + i) * CH
            pltpu.make_async_copy(u.at[pl.ds(base, CH)], u_v.at[b],
                                  sems.at[b]).wait()

        def work(sb, i, b):
            @pl.when(i >= 2)
            def _():
                pltpu.make_async_copy(pay_v.at[b], S_sh.at[ri_v.at[i]],
                                      sems.at[2 + b]).wait()

            def group(g, _):
                s16 = [sv_v[k][i, pl.ds(g * 16, 16)] for k in range(NK)]
                for j in range(16):
                    e = g * 16 + j
                    cs = [s16[k][j] for k in range(NK)]
                    for r in range(4):
                        uv = u_v[b, e, pl.ds(r * 16, 16)]
                        for k in range(NK):
                            pay_v[b, e, pl.ds(64 * k + r * 16, 16)] = \
                                uv * cs[k]
                return 0

            lax.fori_loop(0, CH // 16, group, 0)
            pltpu.async_copy(pay_v.at[b], S_sh.at[ri_v.at[i]], sems.at[2 + b],
                             add=True)

        def sblock(sb, _):
            blk = pl.ds(wid * NCHUNK + sb * SB, SB)
            pltpu.sync_copy(row2.at[blk], ri_v)
            for k in range(NK):
                pltpu.sync_copy(svec[k].at[blk], sv_v[k])
            issue(sb, 0, 0)

            def pair(p, _):
                i0 = 2 * p
                issue(sb, i0 + 1, 1)
                wait(sb, i0, 0)
                work(sb, i0, 0)

                @pl.when(i0 + 2 < SB)
                def _():
                    issue(sb, i0 + 2, 0)
                wait(sb, i0 + 1, 1)
                work(sb, i0 + 1, 1)
                return 0

            lax.fori_loop(0, SB // 2, pair, 0)
            if SB % 2:
                wait(sb, SB - 1, 0)
                work(sb, SB - 1, 0)
            pltpu.make_async_copy(pay_v.at[(SB - 1) % 2],
                                  S_sh.at[ri_v.at[SB - 1]],
                                  sems.at[2 + (SB - 1) % 2]).wait()
            pltpu.make_async_copy(pay_v.at[(SB - 2) % 2],
                                  S_sh.at[ri_v.at[SB - 2]],
                                  sems.at[2 + (SB - 2) % 2]).wait()
            return 0

        lax.fori_loop(0, NCHUNK // SB, sblock, 0)

        plsc.subcore_barrier()
        for q in range(ROWS_PT // ZR):
            rows = pl.ds(sid * ROWS_PT + q * ZR, ZR)
            pltpu.sync_copy(S_sh.at[rows], out.at[cid, rows])

    out = jax.ShapeDtypeStruct((NC, N, W), F32)
    scratch = [
        pltpu.VMEM((SB, CH), I32),
        *([pltpu.VMEM((SB, CH), F32)] * NK),
        pltpu.VMEM((2, CH, 64), F32), pltpu.VMEM((2, CH, W), F32),
        pltpu.SemaphoreType.DMA((4,)),
        pltpu.VMEM_SHARED((N, W), F32), pltpu.VMEM((ZR, W), F32),
    ]
    return pl.kernel(body, out_type=out, mesh=_mesh(), scratch_types=scratch,
                     compiler_params=_SC_PARAMS)


def _geom_tc(rp0, rp1, rp2):
    def fn(p0_ref, p1_ref, p2_ref, rd_ref, s0_ref, s1_ref, s2_ref):
        p0 = p0_ref[...]
        p1 = p1_ref[...]
        p2 = p2_ref[...]
        sxy2 = p0 * p0 + p1 * p1
        rd = jnp.sqrt(sxy2 + p2 * p2)
        theta = jnp.arctan2(p1, p0)
        phi = jnp.arctan2(p2, jnp.sqrt(sxy2))
        rd_ref[...] = rd
        s0_ref[...] = rd * jnp.cos(2.0 * theta)
        s1_ref[...] = rd * jnp.sin(2.0 * theta)
        s2_ref[...] = rd * phi
    shp = jax.ShapeDtypeStruct(rp0.shape, F32)
    return pl.pallas_call(fn, out_shape=(shp,) * 4)(rp0, rp1, rp2)


def _dot(a, b):
    return jax.lax.dot(a, b, preferred_element_type=F32)


def _emb_tc(h, emb_W, emb_b, eW1a, eb1, eW1b):
    N, P = h.shape
    BR = 400
    grid = (N // BR,)

    def fn(h_ref, W_ref, b_ref, W1a_ref, b1_ref, W1b_ref,
           h1_ref, A_ref, B_ref):
        h1 = _dot(h_ref[...], W_ref[...]) + b_ref[...]
        h1_ref[...] = h1
        A_ref[...] = _dot(h1, W1a_ref[...]) + b1_ref[...]
        B_ref[...] = _dot(h1, W1b_ref[...])

    full = lambda s: pl.BlockSpec(s, lambda i: (0, 0))
    rows = lambda w: pl.BlockSpec((BR, w), lambda i: (i, 0))
    return pl.pallas_call(
        fn, grid=grid,
        in_specs=[rows(P), full((P, H)), full((1, H)), full((H, H)),
                  full((1, H)), full((H, H))],
        out_specs=[rows(H)] * 3,
        out_shape=(jax.ShapeDtypeStruct((N, H), F32),) * 3,
    )(h, emb_W, emb_b.reshape(1, H), eW1a, eb1.reshape(1, H), eW1b)


def _node_tc(h, S0, S1, ex0, ex1, eW2, eb2, nW1a, nW1b, nb1, nW2, nb2,
             eW1a, eb1, eW1b):
    N = h.shape[0]
    BR = 400
    grid = (N // BR,)
    WS = S0.shape[1]

    def fn(h_ref, S0_ref, S1_ref, e0_ref, e1_ref, eW2_ref, eb2_ref,
           nW1a_ref, nW1b_ref, nb1_ref, nW2_ref, nb2_ref,
           eW1a_ref, eb1_ref, eW1b_ref, hn_ref, A_ref, B_ref):
        S = S0_ref[...][:, :H] + S1_ref[...][:, :H]
        cnt = e0_ref[...][:, 64:65] + e1_ref[...][:, 64:65]
        agg = _dot(S, eW2_ref[...]) + cnt * eb2_ref[...]
        hh = h_ref[...]
        z = jnp.maximum(
            _dot(hh, nW1a_ref[...]) + _dot(agg, nW1b_ref[...]) + nb1_ref[...],
            0.0)
        hn = hh + _dot(z, nW2_ref[...]) + nb2_ref[...]
        hn_ref[...] = hn
        A_ref[...] = _dot(hn, eW1a_ref[...]) + eb1_ref[...]
        B_ref[...] = _dot(hn, eW1b_ref[...])

    full = lambda s: pl.BlockSpec(s, lambda i: (0, 0))
    rows = lambda w: pl.BlockSpec((BR, w), lambda i: (i, 0))
    return pl.pallas_call(
        fn, grid=grid,
        in_specs=[rows(H), rows(WS), rows(WS), rows(80), rows(80),
                  full((H, H)), full((1, H)), full((H, H)), full((H, H)),
                  full((1, H)), full((H, H)), full((1, H)), full((H, H)),
                  full((1, H)), full((H, H))],
        out_specs=[rows(H)] * 3,
        out_shape=(jax.ShapeDtypeStruct((N, H), F32),) * 3,
    )(h, S0, S1, ex0, ex1, eW2, eb2.reshape(1, H), nW1a, nW1b,
      nb1.reshape(1, H), nW2, nb2.reshape(1, H), eW1a, eb1.reshape(1, H),
      eW1b)


def _w_tc(t3, C, d, wW2, wb2):
    E = t3.shape[0]
    BR = 6400
    grid = (E // BR,)

    def fn(t_ref, C_ref, d_ref, W2_ref, b2_ref, w_ref):
        uu = jnp.maximum(_dot(t_ref[...], C_ref[...]) + d_ref[...], 0.0)
        w_ref[...] = _dot(uu, W2_ref[...]) + b2_ref[...]

    return pl.pallas_call(
        fn, grid=grid,
        in_specs=[pl.BlockSpec((BR, H), lambda i: (i, 0)),
                  pl.BlockSpec((H, H), lambda i: (0, 0)),
                  pl.BlockSpec((1, H), lambda i: (0, 0)),
                  pl.BlockSpec((H, H), lambda i: (0, 0)),
                  pl.BlockSpec((1, H), lambda i: (0, 0))],
        out_specs=pl.BlockSpec((BR, H), lambda i: (i, 0)),
        out_shape=jax.ShapeDtypeStruct((E, H), F32),
    )(t3, C, d, wW2, wb2.reshape(1, H))


def _final_tc(P0, P1, Q0, Q1, ex0, ex1):
    N = P0.shape[0]
    BR = 400
    grid = (N // BR,)

    def fn(P0_ref, P1_ref, Q0_ref, Q1_ref, e0_ref, e1_ref,
           L0_ref, L1_ref, L2_ref, v_ref):
        cnt = jnp.maximum(e0_ref[...][:, 64:65] + e1_ref[...][:, 64:65], 1.0)
        Ls = []
        for k in range(3):
            if k < 2:
                Sk = P0_ref[...][:, k * H:(k + 1) * H] \
                    + P1_ref[...][:, k * H:(k + 1) * H]
            else:
                Sk = Q0_ref[...] + Q1_ref[...]
            Ls.append(Sk / cnt)
        L0_ref[...] = Ls[0]
        L1_ref[...] = Ls[1]
        L2_ref[...] = Ls[2]
        v0 = Ls[0][:, 0:1]
        v1 = Ls[1][:, 0:1]
        inv = 1.0 / jnp.maximum(jnp.sqrt(v0 * v0 + v1 * v1), 1e-12)
        v_ref[...] = jnp.concatenate(
            [v0 * inv, v1 * inv, jnp.zeros((BR, H - 2), F32)], axis=1)

    rows = lambda w: pl.BlockSpec((BR, w), lambda i: (i, 0))
    return pl.pallas_call(
        fn, grid=grid,
        in_specs=[rows(128), rows(128), rows(H), rows(H), rows(80), rows(80)],
        out_specs=[rows(H)] * 4,
        out_shape=(jax.ShapeDtypeStruct((N, H), F32),) * 4,
    )(P0, P1, Q0, Q1, ex0, ex1)


def kernel(h, x, edge_index, emb_W, emb_b, eW1, eb1, eW2, eb2,
           nW1, nb1, nW2, nb2, wW1, wb1, wW2, wb2):
    N, P = h.shape
    E = edge_index.shape[1]
    row = edge_index[0]
    col = edge_index[1]

    eW1a, eW1b, w1c = eW1[:H], eW1[H:2 * H], eW1[2 * H]
    nW1a, nW1b = nW1[:H], nW1[H:2 * H]

    x16 = jnp.concatenate([x, jnp.zeros((N, 13), F32)], axis=1)
    rowc = row.reshape(E // CH, CH)
    colc = col.reshape(E // CH, CH)
    rp3 = _geom_sc(E, N)(x16, rowc, colc)

    g2 = (E // 512, 512)
    rd2, s02, s12, s22 = _geom_tc(rp3[0].reshape(g2), rp3[1].reshape(g2),
                                  rp3[2].reshape(g2))
    gc = (E // CH, CH)
    rdc = rd2.reshape(gc)
    s0c = s02.reshape(gc)
    s1c = s12.reshape(gc)
    s2c = s22.reshape(gc)

    h1, A1, B1 = _emb_tc(h, emb_W, emb_b, eW1a, eb1, eW1b)

    S1t = _layer_sc(E, N, extras=True, write_t3=False)(
        A1, B1, rowc, colc, rdc, w1c)
    h2, A2, B2 = _node_tc(h1, S1t[0], S1t[1], S1t[0], S1t[1], eW2, eb2,
                          nW1a, nW1b, nb1, nW2, nb2, eW1a, eb1, eW1b)

    S2t = _layer_sc(E, N, extras=False, write_t3=False)(
        A2, B2, rowc, colc, rdc, w1c)
    h3, A3, B3 = _node_tc(h2, S2t[0], S2t[1], S1t[0], S1t[1], eW2, eb2,
                          nW1a, nW1b, nb1, nW2, nb2, eW1a, eb1, eW1b)

    t3 = _layer_sc(E, N, extras=False, write_t3=True)(
        A3, B3, rowc, colc, rdc, w1c)

    C = eW2 @ wW1
    d = (eb2 @ wW1 + wb1).reshape(1, H)
    w = _w_tc(t3, C, d, wW2, wb2)

    Pab = _final_sc(E, N, 2)(w, rowc, s0c, s1c)
    Pc = _final_sc(E, N, 1)(w, rowc, s2c)
    L0, L1, L2, vout = _final_tc(Pab[0], Pab[1], Pc[0], Pc[1],
                                 S1t[0], S1t[1])

    v_latent = jnp.stack([L0, L1, L2], axis=-1)
    v = vout[:, :2]
    return (v_latent, x, v)


# speedup vs baseline: 1.0145x; 1.0145x over previous
import jax
import jax.numpy as jnp
from jax import lax
from jax.experimental import pallas as pl
from jax.experimental.pallas import tpu as pltpu
from jax.experimental.pallas import tpu_sc as plsc

F32 = jnp.float32
I32 = jnp.int32
NC, NS, LANES = 2, 16, 16
NW = NC * NS
H = 64
CH = 80


_SC_PARAMS = pltpu.CompilerParams(needs_layout_passes=False,
                                  use_tc_tiling_on_sc=False)


def _mesh():
    return plsc.VectorSubcoreMesh(core_axis_name="c", subcore_axis_name="s",
                                  num_cores=NC, num_subcores=NS)


def _wid():
    return lax.axis_index("s") * NC + lax.axis_index("c")


def _geom_sc(E, N):
    EPT = E // NW
    NCHUNK = EPT // CH
    GROUPS = CH // LANES

    def body(x16, row2, col2, rp0, rp1, rp2, ri_v, ci_v, xr_v, xc_v, rp_v,
             sems):
        wid = _wid()
        pltpu.sync_copy(row2.at[pl.ds(wid * NCHUNK, NCHUNK)], ri_v)
        pltpu.sync_copy(col2.at[pl.ds(wid * NCHUNK, NCHUNK)], ci_v)

        def issue(i, b):
            cr = pltpu.async_copy(x16.at[ri_v.at[i]], xr_v.at[b], sems.at[b])
            cc = pltpu.async_copy(x16.at[ci_v.at[i]], xc_v.at[b],
                                  sems.at[2 + b])
            return cr, cc

        def wait(i, b):
            pltpu.make_async_copy(x16.at[ri_v.at[i]], xr_v.at[b],
                                  sems.at[b]).wait()
            pltpu.make_async_copy(x16.at[ci_v.at[i]], xc_v.at[b],
                                  sems.at[2 + b]).wait()

        def work(i, b):
            for g in range(GROUPS):
                eids = lax.iota(I32, LANES) + g * LANES
                for j in range(3):
                    jv = jnp.full((LANES,), j, I32)
                    d = (plsc.load_gather(xr_v.at[b], [eids, jv])
                         - plsc.load_gather(xc_v.at[b], [eids, jv]))
                    rp_v[j, pl.ds(g * LANES, LANES)] = d
            base = wid * EPT + i * CH
            pltpu.sync_copy(rp_v.at[0], rp0.at[pl.ds(base, CH)])
            pltpu.sync_copy(rp_v.at[1], rp1.at[pl.ds(base, CH)])
            pltpu.sync_copy(rp_v.at[2], rp2.at[pl.ds(base, CH)])

        issue(0, 0)

        def pair(p, _):
            i0 = 2 * p
            issue(i0 + 1, 1)
            wait(i0, 0)
            work(i0, 0)

            @pl.when(i0 + 2 < NCHUNK)
            def _():
                issue(i0 + 2, 0)
            wait(i0 + 1, 1)
            work(i0 + 1, 1)
            return 0

        lax.fori_loop(0, NCHUNK // 2, pair, 0)
        if NCHUNK % 2:
            wait(NCHUNK - 1, 0)
            work(NCHUNK - 1, 0)

    out = tuple(jax.ShapeDtypeStruct((E,), F32) for _ in range(3))
    scratch = [
        pltpu.VMEM((NCHUNK, CH), I32), pltpu.VMEM((NCHUNK, CH), I32),
        pltpu.VMEM((2, CH, 16), F32), pltpu.VMEM((2, CH, 16), F32),
        pltpu.VMEM((3, CH), F32),
        pltpu.SemaphoreType.DMA((4,)),
    ]
    return pl.kernel(body, out_type=out, mesh=_mesh(), scratch_types=scratch,
                     compiler_params=_SC_PARAMS)


def _layer_sc(E, N, extras, write_t3):
    EPT = E // NW
    NCHUNK = EPT // CH
    SB = 25
    DEPTH = 2
    W = 64 + (16 if extras else 0)
    ROWS_PT = N // NS
    ZR = 125

    def body(*refs):
        it = iter(refs)
        A = next(it); Bt = next(it); row2 = next(it); col2 = next(it)
        rd2 = next(it); w1c = next(it)
        out = next(it)
        ri_v = next(it); ci_v = next(it); rd_v = next(it); w1c_v = next(it)
        a_v = next(it); b_v = next(it); pay_v = next(it)
        sems = next(it)
        if not write_t3:
            S_sh = next(it); zbuf = next(it)

        cid = lax.axis_index("c")
        sid = lax.axis_index("s")
        wid = _wid()
        pltpu.sync_copy(w1c, w1c_v)

        if not write_t3:
            def zrow(i, _):
                for r in range(W // 16):
                    zbuf[i, pl.ds(r * 16, 16)] = jnp.zeros((16,), F32)
                return 0
            lax.fori_loop(0, ZR, zrow, 0)
            for q in range(ROWS_PT // ZR):
                pltpu.sync_copy(
                    zbuf, S_sh.at[pl.ds(sid * ROWS_PT + q * ZR, ZR)])
            plsc.subcore_barrier()

        def issue(i, b):
            pltpu.async_copy(A.at[ri_v.at[i]], a_v.at[b], sems.at[b])
            pltpu.async_copy(Bt.at[ci_v.at[i]], b_v.at[b], sems.at[4 + b])

        def wait(i, b):
            pltpu.make_async_copy(A.at[ri_v.at[i]], a_v.at[b],
                                  sems.at[b]).wait()
            pltpu.make_async_copy(Bt.at[ci_v.at[i]], b_v.at[b],
                                  sems.at[4 + b]).wait()

        def out_dst(sb, i, pb):
            if write_t3:
                base = wid * EPT + (sb * SB + i) * CH
                return out.at[pl.ds(base, CH)]
            return S_sh.at[ri_v.at[i]]

        def work(sb, i, b, pb):
            @pl.when(i >= 2)
            def _():
                pltpu.make_async_copy(pay_v.at[pb], out_dst(sb, i, pb),
                                      sems.at[8 + pb]).wait()

            if extras:
                one0 = jnp.where(lax.iota(I32, 16) == 0, 1.0, 0.0)

            def group(g, _):
                rd16 = rd_v[i, pl.ds(g * 16, 16)]
                for j in range(16):
                    rdv = rd16[j]
                    for r in range(4):
                        sl = pl.ds(r * 16, 16)
                        pay_v[pb, g * 16 + j, sl] = jnp.maximum(
                            a_v[b, g * 16 + j, sl] + b_v[b, g * 16 + j, sl]
                            + rdv * w1c_v[sl], 0.0)
                    if extras:
                        pay_v[pb, g * 16 + j, pl.ds(64, 16)] = one0
                return 0

            lax.fori_loop(0, CH // 16, group, 0)
            pltpu.async_copy(pay_v.at[pb], out_dst(sb, i, pb),
                             sems.at[8 + pb], add=not write_t3)

        def sblock(sb, _):
            blk = pl.ds(wid * NCHUNK + sb * SB, SB)
            pltpu.sync_copy(row2.at[blk], ri_v)
            pltpu.sync_copy(col2.at[blk], ci_v)
            pltpu.sync_copy(rd2.at[blk], rd_v)
            for d in range(DEPTH - 1):
                issue(d, d)

            def turn(q, _):
                for l in range(DEPTH):
                    i = DEPTH * q + l

                    @pl.when(i + DEPTH - 1 < SB)
                    def _():
                        issue(i + DEPTH - 1, (l + DEPTH - 1) % DEPTH)
                    wait(i, l)
                    work(sb, i, l, l % 2)
                return 0

            lax.fori_loop(0, SB // DEPTH, turn, 0)
            for i in range(SB - SB % DEPTH, SB):
                wait(i, i % DEPTH)
                work(sb, i, i % DEPTH, i % 2)
            pltpu.make_async_copy(pay_v.at[(SB - 1) % 2],
                                  out_dst(sb, SB - 1, (SB - 1) % 2),
                                  sems.at[8 + (SB - 1) % 2]).wait()
            pltpu.make_async_copy(pay_v.at[(SB - 2) % 2],
                                  out_dst(sb, SB - 2, (SB - 2) % 2),
                                  sems.at[8 + (SB - 2) % 2]).wait()
            return 0

        lax.fori_loop(0, NCHUNK // SB, sblock, 0)

        if not write_t3:
            plsc.subcore_barrier()
            for q in range(ROWS_PT // ZR):
                rows = pl.ds(sid * ROWS_PT + q * ZR, ZR)
                pltpu.sync_copy(S_sh.at[rows], out.at[cid, rows])

    if write_t3:
        out = jax.ShapeDtypeStruct((E, 64), F32)
    else:
        out = jax.ShapeDtypeStruct((NC, N, W), F32)
    scratch = [
        pltpu.VMEM((SB, CH), I32), pltpu.VMEM((SB, CH), I32),
        pltpu.VMEM((SB, CH), F32), pltpu.VMEM((64,), F32),
    ]
    scratch += [
        pltpu.VMEM((DEPTH, CH, 64), F32), pltpu.VMEM((DEPTH, CH, 64), F32),
        pltpu.VMEM((2, CH, W), F32),
        pltpu.SemaphoreType.DMA((10,)),
    ]
    if not write_t3:
        scratch += [pltpu.VMEM_SHARED((N, W), F32), pltpu.VMEM((ZR, W), F32)]
    return pl.kernel(body, out_type=out, mesh=_mesh(), scratch_types=scratch,
                     compiler_params=_SC_PARAMS)


def _final_sc(E, N, NK):
    EPT = E // NW
    NCHUNK = EPT // CH
    SB = 25
    W = 64 * NK
    ROWS_PT = N // NS
    ZR = 25

    def body(*refs):
        it = iter(refs)
        u = next(it)
        row2 = next(it)
        svec = [next(it) for _ in range(NK)]
        out = next(it)
        ri_v = next(it)
        sv_v = [next(it) for _ in range(NK)]
        u_v = next(it)
        pay_v = next(it)
        sems = next(it)
        S_sh = next(it)
        zbuf = next(it)
        cid = lax.axis_index("c")
        sid = lax.axis_index("s")
        wid = _wid()

        def zrow(i, _):
            for r in range(W // 16):
                zbuf[i, pl.ds(r * 16, 16)] = jnp.zeros((16,), F32)
            return 0
        lax.fori_loop(0, ZR, zrow, 0)
        for q in range(ROWS_PT // ZR):
            pltpu.sync_copy(zbuf, S_sh.at[pl.ds(sid * ROWS_PT + q * ZR, ZR)])
        plsc.subcore_barrier()

        def issue(sb, i, b):
            base = wid * EPT + (sb * SB + i) * CH
            pltpu.async_copy(u.at[pl.ds(base, CH)], u_v.at[b], sems.at[b])

        def wait(sb, i, b):
            base = wid * EPT + (sb * SB + i) * CH
            pltpu.make_async_copy(u.at[pl.ds(base, CH)], u_v.at[b],
                                  sems.at[b]).wait()

        def work(sb, i, b):
            @pl.when(i >= 2)
            def _():
                pltpu.make_async_copy(pay_v.at[b], S_sh.at[ri_v.at[i]],
                                      sems.at[2 + b]).wait()

            def group(g, _):
                s16 = [sv_v[k][i, pl.ds(g * 16, 16)] for k in range(NK)]
                for j in range(16):
                    e = g * 16 + j
                    cs = [s16[k][j] for k in range(NK)]
                    for r in range(4):
                        uv = u_v[b, e, pl.ds(r * 16, 16)]
                        for k in range(NK):
                            pay_v[b, e, pl.ds(64 * k + r * 16, 16)] = \
                                uv * cs[k]
                return 0

            lax.fori_loop(0, CH // 16, group, 0)
            pltpu.async_copy(pay_v.at[b], S_sh.at[ri_v.at[i]], sems.at[2 + b],
                             add=True)

        def sblock(sb, _):
            blk = pl.ds(wid * NCHUNK + sb * SB, SB)
            pltpu.sync_copy(row2.at[blk], ri_v)
            for k in range(NK):
                pltpu.sync_copy(svec[k].at[blk], sv_v[k])
            issue(sb, 0, 0)

            def pair(p, _):
                i0 = 2 * p
                issue(sb, i0 + 1, 1)
                wait(sb, i0, 0)
                work(sb, i0, 0)

                @pl.when(i0 + 2 < SB)
                def _():
                    issue(sb, i0 + 2, 0)
                wait(sb, i0 + 1, 1)
                work(sb, i0 + 1, 1)
                return 0

            lax.fori_loop(0, SB // 2, pair, 0)
            if SB % 2:
                wait(sb, SB - 1, 0)
                work(sb, SB - 1, 0)
            pltpu.make_async_copy(pay_v.at[(SB - 1) % 2],
                                  S_sh.at[ri_v.at[SB - 1]],
                                  sems.at[2 + (SB - 1) % 2]).wait()
            pltpu.make_async_copy(pay_v.at[(SB - 2) % 2],
                                  S_sh.at[ri_v.at[SB - 2]],
                                  sems.at[2 + (SB - 2) % 2]).wait()
            return 0

        lax.fori_loop(0, NCHUNK // SB, sblock, 0)

        plsc.subcore_barrier()
        for q in range(ROWS_PT // ZR):
            rows = pl.ds(sid * ROWS_PT + q * ZR, ZR)
            pltpu.sync_copy(S_sh.at[rows], out.at[cid, rows])

    out = jax.ShapeDtypeStruct((NC, N, W), F32)
    scratch = [
        pltpu.VMEM((SB, CH), I32),
        *([pltpu.VMEM((SB, CH), F32)] * NK),
        pltpu.VMEM((2, CH, 64), F32), pltpu.VMEM((2, CH, W), F32),
        pltpu.SemaphoreType.DMA((4,)),
        pltpu.VMEM_SHARED((N, W), F32), pltpu.VMEM((ZR, W), F32),
    ]
    return pl.kernel(body, out_type=out, mesh=_mesh(), scratch_types=scratch,
                     compiler_params=_SC_PARAMS)


def _geom_tc(rp0, rp1, rp2):
    def fn(p0_ref, p1_ref, p2_ref, rd_ref, s0_ref, s1_ref, s2_ref):
        p0 = p0_ref[...]
        p1 = p1_ref[...]
        p2 = p2_ref[...]
        sxy2 = p0 * p0 + p1 * p1
        rd = jnp.sqrt(sxy2 + p2 * p2)
        theta = jnp.arctan2(p1, p0)
        phi = jnp.arctan2(p2, jnp.sqrt(sxy2))
        rd_ref[...] = rd
        s0_ref[...] = rd * jnp.cos(2.0 * theta)
        s1_ref[...] = rd * jnp.sin(2.0 * theta)
        s2_ref[...] = rd * phi
    shp = jax.ShapeDtypeStruct(rp0.shape, F32)
    return pl.pallas_call(fn, out_shape=(shp,) * 4)(rp0, rp1, rp2)


def _dot(a, b):
    return jax.lax.dot(a, b, preferred_element_type=F32)


def _emb_tc(h, emb_W, emb_b, eW1a, eb1, eW1b):
    N, P = h.shape
    BR = 400
    grid = (N // BR,)

    def fn(h_ref, W_ref, b_ref, W1a_ref, b1_ref, W1b_ref,
           h1_ref, A_ref, B_ref):
        h1 = _dot(h_ref[...], W_ref[...]) + b_ref[...]
        h1_ref[...] = h1
        A_ref[...] = _dot(h1, W1a_ref[...]) + b1_ref[...]
        B_ref[...] = _dot(h1, W1b_ref[...])

    full = lambda s: pl.BlockSpec(s, lambda i: (0, 0))
    rows = lambda w: pl.BlockSpec((BR, w), lambda i: (i, 0))
    return pl.pallas_call(
        fn, grid=grid,
        in_specs=[rows(P), full((P, H)), full((1, H)), full((H, H)),
                  full((1, H)), full((H, H))],
        out_specs=[rows(H)] * 3,
        out_shape=(jax.ShapeDtypeStruct((N, H), F32),) * 3,
    )(h, emb_W, emb_b.reshape(1, H), eW1a, eb1.reshape(1, H), eW1b)


def _node_tc(h, S0, S1, ex0, ex1, eW2, eb2, nW1a, nW1b, nb1, nW2, nb2,
             eW1a, eb1, eW1b):
    N = h.shape[0]
    BR = 400
    grid = (N // BR,)
    WS = S0.shape[1]

    def fn(h_ref, S0_ref, S1_ref, e0_ref, e1_ref, eW2_ref, eb2_ref,
           nW1a_ref, nW1b_ref, nb1_ref, nW2_ref, nb2_ref,
           eW1a_ref, eb1_ref, eW1b_ref, hn_ref, A_ref, B_ref):
        S = S0_ref[...][:, :H] + S1_ref[...][:, :H]
        cnt = e0_ref[...][:, 64:65] + e1_ref[...][:, 64:65]
        agg = _dot(S, eW2_ref[...]) + cnt * eb2_ref[...]
        hh = h_ref[...]
        z = jnp.maximum(
            _dot(hh, nW1a_ref[...]) + _dot(agg, nW1b_ref[...]) + nb1_ref[...],
            0.0)
        hn = hh + _dot(z, nW2_ref[...]) + nb2_ref[...]
        hn_ref[...] = hn
        A_ref[...] = _dot(hn, eW1a_ref[...]) + eb1_ref[...]
        B_ref[...] = _dot(hn, eW1b_ref[...])

    full = lambda s: pl.BlockSpec(s, lambda i: (0, 0))
    rows = lambda w: pl.BlockSpec((BR, w), lambda i: (i, 0))
    return pl.pallas_call(
        fn, grid=grid,
        in_specs=[rows(H), rows(WS), rows(WS), rows(80), rows(80),
                  full((H, H)), full((1, H)), full((H, H)), full((H, H)),
                  full((1, H)), full((H, H)), full((1, H)), full((H, H)),
                  full((1, H)), full((H, H))],
        out_specs=[rows(H)] * 3,
        out_shape=(jax.ShapeDtypeStruct((N, H), F32),) * 3,
    )(h, S0, S1, ex0, ex1, eW2, eb2.reshape(1, H), nW1a, nW1b,
      nb1.reshape(1, H), nW2, nb2.reshape(1, H), eW1a, eb1.reshape(1, H),
      eW1b)


def _w_tc(t3, C, d, wW2, wb2):
    E = t3.shape[0]
    BR = 6400
    grid = (E // BR,)

    def fn(t_ref, C_ref, d_ref, W2_ref, b2_ref, w_ref):
        uu = jnp.maximum(_dot(t_ref[...], C_ref[...]) + d_ref[...], 0.0)
        w_ref[...] = _dot(uu, W2_ref[...]) + b2_ref[...]

    return pl.pallas_call(
        fn, grid=grid,
        in_specs=[pl.BlockSpec((BR, H), lambda i: (i, 0)),
                  pl.BlockSpec((H, H), lambda i: (0, 0)),
                  pl.BlockSpec((1, H), lambda i: (0, 0)),
                  pl.BlockSpec((H, H), lambda i: (0, 0)),
                  pl.BlockSpec((1, H), lambda i: (0, 0))],
        out_specs=pl.BlockSpec((BR, H), lambda i: (i, 0)),
        out_shape=jax.ShapeDtypeStruct((E, H), F32),
    )(t3, C, d, wW2, wb2.reshape(1, H))


def _final_tc(P0, P1, Q0, Q1, ex0, ex1):
    N = P0.shape[0]
    BR = 400
    grid = (N // BR,)

    def fn(P0_ref, P1_ref, Q0_ref, Q1_ref, e0_ref, e1_ref,
           L0_ref, L1_ref, L2_ref, v_ref):
        cnt = jnp.maximum(e0_ref[...][:, 64:65] + e1_ref[...][:, 64:65], 1.0)
        Ls = []
        for k in range(3):
            if k < 2:
                Sk = P0_ref[...][:, k * H:(k + 1) * H] \
                    + P1_ref[...][:, k * H:(k + 1) * H]
            else:
                Sk = Q0_ref[...] + Q1_ref[...]
            Ls.append(Sk / cnt)
        L0_ref[...] = Ls[0]
        L1_ref[...] = Ls[1]
        L2_ref[...] = Ls[2]
        v0 = Ls[0][:, 0:1]
        v1 = Ls[1][:, 0:1]
        inv = 1.0 / jnp.maximum(jnp.sqrt(v0 * v0 + v1 * v1), 1e-12)
        v_ref[...] = jnp.concatenate(
            [v0 * inv, v1 * inv, jnp.zeros((BR, H - 2), F32)], axis=1)

    rows = lambda w: pl.BlockSpec((BR, w), lambda i: (i, 0))
    return pl.pallas_call(
        fn, grid=grid,
        in_specs=[rows(128), rows(128), rows(H), rows(H), rows(80), rows(80)],
        out_specs=[rows(H)] * 4,
        out_shape=(jax.ShapeDtypeStruct((N, H), F32),) * 4,
    )(P0, P1, Q0, Q1, ex0, ex1)


def kernel(h, x, edge_index, emb_W, emb_b, eW1, eb1, eW2, eb2,
           nW1, nb1, nW2, nb2, wW1, wb1, wW2, wb2):
    N, P = h.shape
    E = edge_index.shape[1]
    row = edge_index[0]
    col = edge_index[1]

    eW1a, eW1b, w1c = eW1[:H], eW1[H:2 * H], eW1[2 * H]
    nW1a, nW1b = nW1[:H], nW1[H:2 * H]

    x16 = jnp.concatenate([x, jnp.zeros((N, 13), F32)], axis=1)
    rowc = row.reshape(E // CH, CH)
    colc = col.reshape(E // CH, CH)
    rp0, rp1, rp2 = _geom_sc(E, N)(x16, rowc, colc)

    g2 = (E // 512, 512)
    rd2, s02, s12, s22 = _geom_tc(rp0.reshape(g2), rp1.reshape(g2),
                                  rp2.reshape(g2))
    gc = (E // CH, CH)
    rdc = rd2.reshape(gc)
    s0c = s02.reshape(gc)
    s1c = s12.reshape(gc)
    s2c = s22.reshape(gc)

    h1, A1, B1 = _emb_tc(h, emb_W, emb_b, eW1a, eb1, eW1b)

    S1t = _layer_sc(E, N, extras=True, write_t3=False)(
        A1, B1, rowc, colc, rdc, w1c)
    h2, A2, B2 = _node_tc(h1, S1t[0], S1t[1], S1t[0], S1t[1], eW2, eb2,
                          nW1a, nW1b, nb1, nW2, nb2, eW1a, eb1, eW1b)

    S2t = _layer_sc(E, N, extras=False, write_t3=False)(
        A2, B2, rowc, colc, rdc, w1c)
    h3, A3, B3 = _node_tc(h2, S2t[0], S2t[1], S1t[0], S1t[1], eW2, eb2,
                          nW1a, nW1b, nb1, nW2, nb2, eW1a, eb1, eW1b)

    t3 = _layer_sc(E, N, extras=False, write_t3=True)(
        A3, B3, rowc, colc, rdc, w1c)

    C = eW2 @ wW1
    d = (eb2 @ wW1 + wb1).reshape(1, H)
    w = _w_tc(t3, C, d, wW2, wb2)

    Pab = _final_sc(E, N, 2)(w, rowc, s0c, s1c)
    Pc = _final_sc(E, N, 1)(w, rowc, s2c)
    L0, L1, L2, vout = _final_tc(Pab[0], Pab[1], Pc[0], Pc[1],
                                 S1t[0], S1t[1])

    v_latent = jnp.stack([L0, L1, L2], axis=-1)
    v = vout[:, :2]
    return (v_latent, x, v)
